# Initial kernel scaffold; baseline (speedup 1.0000x reference)
import jax
import jax.numpy as jnp
from jax.experimental import pallas as pl


def kernel(x, g, w_start, b_start, w_cat, b_cat, W1, al1, ar1, bias1, W2, al2, ar2, bias2):
    raise NotImplementedError("write your pallas kernel here")


# speedup vs baseline: 2.1494x; 2.1494x over previous
import functools

import jax
import jax.numpy as jnp
from jax import lax
from jax.experimental import pallas as pl
from jax.experimental.pallas import tpu as pltpu

B, C, N, T = 2, 16, 1000, 10
E, H = 16, 8
D = E * T


def _conv_kernel(x_ref, ws_ref, wc_ref, bs_ref, bc_ref, out_ref):
    x = x_ref[...]
    x1 = jnp.dot(x, ws_ref[...], preferred_element_type=jnp.float32) + bs_ref[...]
    x2 = jnp.dot(x, wc_ref[...], preferred_element_type=jnp.float32) + bc_ref[...]
    x2 = jnp.where(x2 >= 0, x2, 0.01 * x2)
    out_ref[...] = x1 + x2


def _gat_kernel(with_res, *refs):
    if with_res:
        h_ref, w_ref, al_ref, ar_ref, bias_ref, gt_ref, res_ref, out_ref = refs
    else:
        h_ref, w_ref, al_ref, ar_ref, bias_ref, gt_ref, out_ref = refs
        res_ref = None
    hh = pl.program_id(1)

    h_b = h_ref[0]
    w = w_ref[0]
    feat = jnp.dot(h_b, w, preferred_element_type=jnp.float32)

    al = al_ref[0]
    ar = ar_ref[0]
    dnum = (((1,), (1,)), ((), ()))
    el_row = lax.dot_general(al, feat, dnum, preferred_element_type=jnp.float32)
    er_col = lax.dot_general(feat, ar, dnum, preferred_element_type=jnp.float32)

    s = er_col + el_row
    s = jnp.where(s >= 0, s, 0.2 * s)
    mask = gt_ref[...] != 0.0
    s = jnp.where(mask, s, -1e30)
    m = jnp.max(s, axis=1, keepdims=True)
    p = jnp.exp(s - m)
    z = jnp.sum(p, axis=1, keepdims=True)
    alpha = p / z
    alpha = jnp.where(mask, alpha, 0.0)

    rst = jnp.dot(alpha, feat, preferred_element_type=jnp.float32)
    rst = rst + bias_ref[0]
    rst = jnp.where(rst > 0, rst, jnp.exp(rst) - 1.0)
    acc = rst * (1.0 / H)

    @pl.when(hh == 0)
    def _init():
        if res_ref is None:
            out_ref[0] = acc
        else:
            out_ref[0] = res_ref[0] + acc

    @pl.when(hh != 0)
    def _acc():
        out_ref[0] = out_ref[0] + acc


def _gat_layer(h, W, al, ar, bias, gt, res=None):
    W_r = W.reshape(D, H, D).transpose(1, 0, 2)
    bias_r = bias.reshape(H, 1, D)
    al_r = al.reshape(H, 1, D)
    ar_r = ar.reshape(H, 1, D)
    inputs = [h, W_r, al_r, ar_r, bias_r, gt]
    in_specs = [
        pl.BlockSpec((1, N, D), lambda b, hh: (b, 0, 0)),
        pl.BlockSpec((1, D, D), lambda b, hh: (hh, 0, 0)),
        pl.BlockSpec((1, 1, D), lambda b, hh: (hh, 0, 0)),
        pl.BlockSpec((1, 1, D), lambda b, hh: (hh, 0, 0)),
        pl.BlockSpec((1, 1, D), lambda b, hh: (hh, 0, 0)),
        pl.BlockSpec((N, N), lambda b, hh: (0, 0)),
    ]
    if res is not None:
        inputs.append(res)
        in_specs.append(pl.BlockSpec((1, N, D), lambda b, hh: (b, 0, 0)))
    return pl.pallas_call(
        functools.partial(_gat_kernel, res is not None),
        grid=(B, H),
        in_specs=in_specs,
        out_specs=pl.BlockSpec((1, N, D), lambda b, hh: (b, 0, 0)),
        out_shape=jax.ShapeDtypeStruct((B, N, D), jnp.float32),
    )(*inputs)


def kernel(x, g, w_start, b_start, w_cat, b_cat, W1, al1, ar1, bias1, W2, al2, ar2, bias2):
    X = x.transpose(0, 2, 1, 3).reshape(B * N, C * T)
    eye_t = jnp.eye(T, dtype=jnp.float32)
    Wbs = jnp.einsum('ec,tu->cteu', w_start, eye_t).reshape(C * T, E * T)
    Wbc = jnp.einsum('ec,tu->cteu', w_cat, eye_t).reshape(C * T, E * T)
    bs = jnp.repeat(b_start, T).reshape(1, E * T)
    bc = jnp.repeat(b_cat, T).reshape(1, E * T)
    gt = g.T

    xs_flat = pl.pallas_call(
        _conv_kernel,
        out_shape=jax.ShapeDtypeStruct((B * N, E * T), jnp.float32),
    )(X, Wbs, Wbc, bs, bc)

    h0 = xs_flat.reshape(B, N, D)
    h1 = _gat_layer(h0, W1, al1, ar1, bias1, gt)
    h2 = _gat_layer(h1, W2, al2, ar2, bias2, gt, res=h0)

    out = h2.reshape(B, N, E, T).transpose(0, 2, 1, 3)
    return out
